# baseline (speedup 1.0000x reference)
import functools

import jax
import jax.numpy as jnp
from jax import lax
from jax.experimental import pallas as pl
from jax.experimental.pallas import tpu as pltpu
from jax.experimental.pallas import tpu_sc as plsc

_NC, _NS, _L = 2, 16, 16
_NW = _NC * _NS
_T = 16
_BLK = 1024


def _tc_body(thr_ref, m0_ref, x0_ref, x1_ref, o0_ref):
    t = thr_ref[0]
    mt0 = m0_ref[0].T
    for s in range(8):
        k0 = lax.slice(mt0, (0, s), (128, s + 1)) >= t
        rows = pl.ds(s * 128, 128)
        a = x0_ref[rows, :]
        b = x1_ref[rows, :]
        o0_ref[rows, :] = jnp.where(k0, a, b)


def _tc_out0(thr, m0, x0f, x1f, M, C):
    nblk = M // _BLK
    return pl.pallas_call(
        _tc_body,
        grid=(nblk,),
        in_specs=[
            pl.BlockSpec(memory_space=pltpu.SMEM),
            pl.BlockSpec((1, 8, 128), lambda i: (i, 0, 0)),
            pl.BlockSpec((_BLK, C), lambda i: (i, 0)),
            pl.BlockSpec((_BLK, C), lambda i: (i, 0)),
        ],
        out_specs=pl.BlockSpec((_BLK, C), lambda i: (i, 0)),
        out_shape=jax.ShapeDtypeStruct((M, C), jnp.float32),
    )(thr, m0.reshape(nblk, 8, 128), x0f, x1f)


def _make_sc_call(M, C):
    R = M // _NW
    n_pairs = R // (2 * _T)
    mesh = plsc.VectorSubcoreMesh(core_axis_name="c", subcore_axis_name="s")

    @functools.partial(
        pl.kernel,
        out_type=jax.ShapeDtypeStruct((M, C), jnp.float32),
        mesh=mesh,
        scratch_types=[
            pltpu.VMEM((_T, C), jnp.float32),
            pltpu.VMEM((_T, C), jnp.float32),
            pltpu.VMEM((_T, C), jnp.float32),
            pltpu.VMEM((_T, C), jnp.float32),
            pltpu.VMEM((_T, C), jnp.float32),
            pltpu.VMEM((_T, C), jnp.float32),
            pltpu.VMEM((R,), jnp.float32),
            pltpu.VMEM((_L,), jnp.float32),
            pltpu.SemaphoreType.DMA,
            pltpu.SemaphoreType.DMA,
            pltpu.SemaphoreType.DMA,
            pltpu.SemaphoreType.DMA,
        ],
    )
    def sc_call(thr_hbm, m1_hbm, x0_hbm, x1_hbm, o1_hbm,
                x0c0, x0c1, x1c0, x1c1, o1c0, o1c1,
                m1all, thr_v, sem_in0, sem_in1, sem_out0, sem_out1):
        wid = lax.axis_index("s") * _NC + lax.axis_index("c")
        base_row = wid * R
        pltpu.sync_copy(thr_hbm, thr_v)
        pltpu.sync_copy(m1_hbm.at[pl.ds(base_row, R)], m1all)
        thrv = thr_v[...]

        def start_in(c, xb0, xb1, sem):
            row = base_row + c * _T
            pltpu.make_async_copy(x0_hbm.at[pl.ds(row, _T)], xb0, sem).start()
            pltpu.make_async_copy(x1_hbm.at[pl.ds(row, _T)], xb1, sem).start()

        def wait_in(xb0, xb1, sem):
            pltpu.make_async_copy(x0_hbm.at[pl.ds(0, _T)], xb0, sem).wait()
            pltpu.make_async_copy(x1_hbm.at[pl.ds(0, _T)], xb1, sem).wait()

        def start_out(c, ob, sem):
            row = base_row + c * _T
            pltpu.make_async_copy(ob, o1_hbm.at[pl.ds(row, _T)], sem).start()

        def wait_out(ob, sem):
            pltpu.make_async_copy(ob, o1_hbm.at[pl.ds(0, _T)], sem).wait()

        def compute(mv1, xa, xb, ob):
            for t in range(_T):
                kv1 = jnp.broadcast_to(mv1[t], (_L,)) >= thrv
                xat = xa.at[t]
                xbt = xb.at[t]
                obt = ob.at[t]

                @plsc.parallel_loop(0, C, step=_L, unroll=8)
                def jbody(off, kv1=kv1, xat=xat, xbt=xbt, obt=obt):
                    sl = pl.ds(off, _L)
                    a = xat[sl]
                    b = xbt[sl]
                    obt[sl] = jnp.where(kv1, b, a)

        start_in(0, x0c0, x1c0, sem_in0)

        def pair_body(k, carry):
            mv1a = m1all[pl.ds((2 * k) * _T, _T)]
            start_in(2 * k + 1, x0c1, x1c1, sem_in1)
            wait_in(x0c0, x1c0, sem_in0)

            @pl.when(k > 0)
            def _():
                wait_out(o1c0, sem_out0)

            compute(mv1a, x0c0, x1c0, o1c0)
            start_out(2 * k, o1c0, sem_out0)

            mv1b = m1all[pl.ds((2 * k + 1) * _T, _T)]

            @pl.when(k < n_pairs - 1)
            def _():
                start_in(2 * k + 2, x0c0, x1c0, sem_in0)

            wait_in(x0c1, x1c1, sem_in1)

            @pl.when(k > 0)
            def _():
                wait_out(o1c1, sem_out1)

            compute(mv1b, x0c1, x1c1, o1c1)
            start_out(2 * k + 1, o1c1, sem_out1)
            return carry

        lax.fori_loop(0, n_pairs, pair_body, 0)
        wait_out(o1c0, sem_out0)
        wait_out(o1c1, sem_out1)

    return sc_call


def kernel(x0, x1, mask0, mask1, mask_threshold):
    B, N, C = x0.shape
    M = B * N
    x0f = x0.reshape(M, C)
    x1f = x1.reshape(M, C)
    thr = jnp.full((_L,), mask_threshold, jnp.float32)
    o1 = _make_sc_call(M, C)(thr, mask1.reshape(M), x0f, x1f)
    o0 = _tc_out0(thr[:1], mask0.reshape(M), x0f, x1f, M, C)
    return o0.reshape(B, N, C), o1.reshape(B, N, C)


# speedup vs baseline: 1.3133x; 1.3133x over previous
import functools

import jax
import jax.numpy as jnp
from jax import lax
from jax.experimental import pallas as pl
from jax.experimental.pallas import tpu as pltpu
from jax.experimental.pallas import tpu_sc as plsc

_NC, _NS, _L = 2, 16, 16
_NW = _NC * _NS
_T = 8


def _make_sc_call(M, C):
    R = M // _NW
    n_chunks = R // _T
    n_quads = n_chunks // 4
    mesh = plsc.VectorSubcoreMesh(core_axis_name="c", subcore_axis_name="s")

    @functools.partial(
        pl.kernel,
        out_type=[
            jax.ShapeDtypeStruct((M, C), jnp.float32),
            jax.ShapeDtypeStruct((M, C), jnp.float32),
        ],
        mesh=mesh,
        scratch_types=[
            pltpu.VMEM((4, _T, C), jnp.float32),
            pltpu.VMEM((4, _T, C), jnp.float32),
            pltpu.VMEM((2, _T, C), jnp.float32),
            pltpu.VMEM((2, _T, C), jnp.float32),
            pltpu.VMEM((R,), jnp.float32),
            pltpu.VMEM((R,), jnp.float32),
            pltpu.VMEM((_L,), jnp.float32),
            pltpu.SemaphoreType.DMA,
            pltpu.SemaphoreType.DMA,
            pltpu.SemaphoreType.DMA,
            pltpu.SemaphoreType.DMA,
            pltpu.SemaphoreType.DMA,
            pltpu.SemaphoreType.DMA,
        ],
    )
    def sc_call(thr_hbm, m0_hbm, m1_hbm, x0_hbm, x1_hbm, o0_hbm, o1_hbm,
                x0c, x1c, o0c, o1c, m0all, m1all, thr_v,
                sem_in0, sem_in1, sem_in2, sem_in3, sem_out0, sem_out1):
        sem_in = (sem_in0, sem_in1, sem_in2, sem_in3)
        sem_out = (sem_out0, sem_out1)
        wid = lax.axis_index("s") * _NC + lax.axis_index("c")
        base_row = wid * R
        pltpu.sync_copy(thr_hbm, thr_v)
        pltpu.sync_copy(m0_hbm.at[pl.ds(base_row, R)], m0all)
        pltpu.sync_copy(m1_hbm.at[pl.ds(base_row, R)], m1all)
        thrv = thr_v[...]

        def start_in(c, i):
            row = base_row + c * _T
            s = sem_in[i]
            pltpu.make_async_copy(
                x0_hbm.at[pl.ds(row, _T)], x0c.at[i], s).start()
            pltpu.make_async_copy(
                x1_hbm.at[pl.ds(row, _T)], x1c.at[i], s).start()

        def wait_in(i):
            s = sem_in[i]
            pltpu.make_async_copy(
                x0_hbm.at[pl.ds(0, _T)], x0c.at[i], s).wait()
            pltpu.make_async_copy(
                x1_hbm.at[pl.ds(0, _T)], x1c.at[i], s).wait()

        def start_out(c, i):
            row = base_row + c * _T
            s = sem_out[i]
            pltpu.make_async_copy(
                o0c.at[i], o0_hbm.at[pl.ds(row, _T)], s).start()
            pltpu.make_async_copy(
                o1c.at[i], o1_hbm.at[pl.ds(row, _T)], s).start()

        def wait_out(i):
            s = sem_out[i]
            pltpu.make_async_copy(
                o0c.at[i], o0_hbm.at[pl.ds(0, _T)], s).wait()
            pltpu.make_async_copy(
                o1c.at[i], o1_hbm.at[pl.ds(0, _T)], s).wait()

        def compute(mv0, mv1, lane0, i, oi):
            xa = x0c.at[i]
            xb = x1c.at[i]
            oa = o0c.at[oi]
            ob = o1c.at[oi]
            for t in range(_T):
                kv0 = jnp.broadcast_to(mv0[lane0 + t], (_L,)) >= thrv
                kv1 = jnp.broadcast_to(mv1[lane0 + t], (_L,)) >= thrv
                xat = xa.at[t]
                xbt = xb.at[t]
                oat = oa.at[t]
                obt = ob.at[t]

                @plsc.parallel_loop(0, C, step=_L, unroll=8)
                def jbody(off, kv0=kv0, kv1=kv1, xat=xat, xbt=xbt,
                          oat=oat, obt=obt):
                    sl = pl.ds(off, _L)
                    a = xat[sl]
                    b = xbt[sl]
                    oat[sl] = jnp.where(kv0, a, b)
                    obt[sl] = jnp.where(kv1, b, a)

        start_in(0, 0)
        start_in(1, 1)
        start_in(2, 2)

        def quad_body(k, carry):
            mva0 = m0all[pl.ds(4 * k * _T, 16)]
            mva1 = m1all[pl.ds(4 * k * _T, 16)]
            mvb0 = m0all[pl.ds((4 * k + 2) * _T, 16)]
            mvb1 = m1all[pl.ds((4 * k + 2) * _T, 16)]
            for i in range(4):
                c = 4 * k + i
                if i == 0:
                    start_in(4 * k + 3, 3)
                else:
                    @pl.when(k < n_quads - 1)
                    def _(i=i):
                        start_in(4 * k + 3 + i, (3 + i) % 4)
                wait_in(i)
                if i < 2:
                    @pl.when(k > 0)
                    def _(i=i):
                        wait_out(i % 2)
                else:
                    wait_out(i % 2)
                mv0, mv1 = (mva0, mva1) if i < 2 else (mvb0, mvb1)
                compute(mv0, mv1, (i % 2) * _T, i, i % 2)
                start_out(c, i % 2)
            return carry

        lax.fori_loop(0, n_quads, quad_body, 0)
        wait_out(0)
        wait_out(1)

    return sc_call


def kernel(x0, x1, mask0, mask1, mask_threshold):
    B, N, C = x0.shape
    M = B * N
    x0f = x0.reshape(M, C)
    x1f = x1.reshape(M, C)
    m0 = mask0.reshape(M)
    m1 = mask1.reshape(M)
    thr = jnp.full((_L,), mask_threshold, jnp.float32)
    o0, o1 = _make_sc_call(M, C)(thr, m0, m1, x0f, x1f)
    return o0.reshape(B, N, C), o1.reshape(B, N, C)
